# baseline (speedup 1.0000x reference)
import functools

import jax
import jax.numpy as jnp
from jax import lax
from jax.experimental import pallas as pl
from jax.experimental.pallas import tpu as pltpu
from jax.experimental.pallas import tpu_sc as plsc

_D = 64
_SCALE = float(_D) ** 0.5
_NW = 32
_CHUNK = 128
_LANES = 16


def _make_kernel(n_b, n_j):
    n_bhi = n_b // _CHUNK
    nchunk = n_j * n_bhi // _NW
    mesh = plsc.VectorSubcoreMesh(core_axis_name="c", subcore_axis_name="s")

    @functools.partial(
        pl.kernel,
        mesh=mesh,
        out_type=jax.ShapeDtypeStruct((n_j, _D // 8, n_bhi, 8, _CHUNK),
                                      jnp.float32),
        scratch_types=[
            pltpu.VMEM((nchunk, _CHUNK), jnp.int32),
            pltpu.VMEM((_CHUNK, _D), jnp.float32),
            pltpu.VMEM((_CHUNK, _D), jnp.float32),
            pltpu.VMEM((_D // 8, 8, _CHUNK), jnp.float32),
            pltpu.VMEM((_D // 8, 8, _CHUNK), jnp.float32),
            pltpu.SemaphoreType.DMA,
            pltpu.SemaphoreType.DMA,
            pltpu.SemaphoreType.DMA,
            pltpu.SemaphoreType.DMA,
        ],
        compiler_params=pltpu.CompilerParams(use_tc_tiling_on_sc=False,
                                             needs_layout_passes=False),
    )
    def emb(idx_hbm, table_hbm, out_hbm, idx_v, rows_a, rows_b,
            stage_a, stage_b, gsem_a, gsem_b, osem_a, osem_b):
        wid = lax.axis_index("s") * 2 + lax.axis_index("c")
        base_c = wid * nchunk
        pltpu.sync_copy(idx_hbm.at[pl.ds(base_c, nchunk)], idx_v)

        iota = lax.iota(jnp.int32, _LANES)

        def transpose_scale(rows, stage):
            def tile_body(d_hi, carry):
                for d_lo in range(8):
                    d_splat = jnp.broadcast_to(d_hi * 8 + d_lo, (_LANES,))
                    for k in range(8):
                        v = plsc.load_gather(
                            rows, [iota + k * _LANES, d_splat])
                        stage[d_hi, d_lo, pl.ds(k * _LANES, _LANES)] = (
                            v * _SCALE)
                return carry
            lax.fori_loop(0, 8, tile_body, 0)

        def start_gather(c_local, rows, sem):
            return pltpu.async_copy(
                table_hbm.at[idx_v.at[c_local]], rows, sem)

        def start_out(c_local, stage, sem):
            c = base_c + c_local
            j = c // n_bhi
            b_hi = lax.rem(c, n_bhi)
            return pltpu.async_copy(out_hbm.at[j, :, b_hi], stage, sem)

        def wait_gather(rows, sem):
            pltpu.make_async_copy(table_hbm.at[idx_v.at[0]], rows, sem).wait()

        def wait_out(c_hint, stage, sem):
            pltpu.make_async_copy(stage, out_hbm.at[0, :, 0], sem).wait()

        def body(t, carry):
            start_gather(t, rows_a, gsem_a).wait()
            transpose_scale(rows_a, stage_a)
            c = base_c + t
            j = c // n_bhi
            b_hi = lax.rem(c, n_bhi)
            pltpu.sync_copy(stage_a, out_hbm.at[j, :, b_hi])
            return carry

        lax.fori_loop(0, nchunk, body, 0)

    return emb


@jax.jit
def kernel(x, table):
    n_b, n_j = x.shape
    idx2 = x.T.reshape(n_j * (n_b // _CHUNK), _CHUNK)
    out5 = _make_kernel(n_b, n_j)(idx2, table)
    out = out5.transpose(2, 4, 0, 1, 3).reshape(n_b, n_j, _D)
    return out


# speedup vs baseline: 1.1363x; 1.1363x over previous
import functools

import jax
import jax.numpy as jnp
from jax import lax
from jax.experimental import pallas as pl
from jax.experimental.pallas import tpu as pltpu
from jax.experimental.pallas import tpu_sc as plsc

_D = 64
_SCALE = float(_D) ** 0.5
_NW = 32
_CHUNK = 128
_LANES = 16


def _make_kernel(n_b, n_j):
    n_bhi = n_b // _CHUNK
    nchunk = n_j * n_bhi // _NW
    mesh = plsc.VectorSubcoreMesh(core_axis_name="c", subcore_axis_name="s")

    @functools.partial(
        pl.kernel,
        mesh=mesh,
        out_type=jax.ShapeDtypeStruct((n_j, _D // 8, n_bhi, 8, _CHUNK),
                                      jnp.float32),
        scratch_types=[
            pltpu.VMEM((nchunk, _CHUNK), jnp.int32),
            pltpu.VMEM((_CHUNK, _D), jnp.float32),
            pltpu.VMEM((_CHUNK, _D), jnp.float32),
            pltpu.VMEM((_D // 8, 8, _CHUNK), jnp.float32),
            pltpu.VMEM((_D // 8, 8, _CHUNK), jnp.float32),
            pltpu.SemaphoreType.DMA,
            pltpu.SemaphoreType.DMA,
            pltpu.SemaphoreType.DMA,
            pltpu.SemaphoreType.DMA,
        ],
        compiler_params=pltpu.CompilerParams(use_tc_tiling_on_sc=False,
                                             needs_layout_passes=False),
    )
    def emb(idx_hbm, table_hbm, out_hbm, idx_v, rows_a, rows_b,
            stage_a, stage_b, gsem_a, gsem_b, osem_a, osem_b):
        wid = lax.axis_index("s") * 2 + lax.axis_index("c")
        base_c = wid * nchunk
        pltpu.sync_copy(idx_hbm.at[pl.ds(base_c, nchunk)], idx_v)

        iota = lax.iota(jnp.int32, _LANES)

        def transpose_scale(rows, stage):
            def tile_body(d_hi, carry):
                for d_lo in range(8):
                    d_splat = jnp.broadcast_to(d_hi * 8 + d_lo, (_LANES,))
                    for k in range(8):
                        v = plsc.load_gather(
                            rows, [iota + k * _LANES, d_splat])
                        stage[d_hi, d_lo, pl.ds(k * _LANES, _LANES)] = (
                            v * _SCALE)
                return carry
            lax.fori_loop(0, 8, tile_body, 0)

        def start_gather(c_local, rows, sem):
            return pltpu.async_copy(
                table_hbm.at[idx_v.at[c_local]], rows, sem)

        def start_out(c_local, stage, sem):
            c = base_c + c_local
            j = c // n_bhi
            b_hi = lax.rem(c, n_bhi)
            return pltpu.async_copy(stage, out_hbm.at[j, :, b_hi], sem)

        def wait_gather(rows, sem):
            pltpu.make_async_copy(table_hbm.at[pl.ds(0, _CHUNK)], rows,
                                  sem).wait()

        def wait_out(stage, sem):
            pltpu.make_async_copy(stage, out_hbm.at[0, :, 0], sem).wait()

        start_gather(0, rows_a, gsem_a)

        def body(t, carry):
            a = 2 * t
            b = 2 * t + 1
            wait_gather(rows_a, gsem_a)
            start_gather(b, rows_b, gsem_b)

            @pl.when(t > 0)
            def _():
                wait_out(stage_a, osem_a)
            transpose_scale(rows_a, stage_a)
            start_out(a, stage_a, osem_a)

            wait_gather(rows_b, gsem_b)

            @pl.when(t < nchunk // 2 - 1)
            def _():
                start_gather(b + 1, rows_a, gsem_a)

            @pl.when(t > 0)
            def _():
                wait_out(stage_b, osem_b)
            transpose_scale(rows_b, stage_b)
            start_out(b, stage_b, osem_b)
            return carry

        lax.fori_loop(0, nchunk // 2, body, 0)
        wait_out(stage_a, osem_a)
        wait_out(stage_b, osem_b)

    return emb


@jax.jit
def kernel(x, table):
    n_b, n_j = x.shape
    idx2 = x.T.reshape(n_j * (n_b // _CHUNK), _CHUNK)
    out5 = _make_kernel(n_b, n_j)(idx2, table)
    out = out5.transpose(2, 4, 0, 1, 3).reshape(n_b, n_j, _D)
    return out


# speedup vs baseline: 1.4703x; 1.2939x over previous
import functools

import jax
import jax.numpy as jnp
from jax import lax
from jax.experimental import pallas as pl
from jax.experimental.pallas import tpu as pltpu
from jax.experimental.pallas import tpu_sc as plsc

_D = 64
_SCALE = float(_D) ** 0.5
_NW = 32
_CHUNK = 128
_LANES = 16


def _make_kernel(n_b, n_j):
    n_bhi = n_b // _CHUNK
    nchunk = n_j * n_bhi // _NW
    mesh = plsc.VectorSubcoreMesh(core_axis_name="c", subcore_axis_name="s")

    @functools.partial(
        pl.kernel,
        mesh=mesh,
        out_type=jax.ShapeDtypeStruct((n_j, _D // 8, n_bhi, 8, _CHUNK),
                                      jnp.float32),
        scratch_types=[
            pltpu.VMEM((nchunk, _CHUNK), jnp.int32),
            pltpu.VMEM((_CHUNK, _D), jnp.float32),
            pltpu.VMEM((_CHUNK, _D), jnp.float32),
            pltpu.VMEM((_D // 8, 8, _CHUNK), jnp.float32),
            pltpu.VMEM((_D // 8, 8, _CHUNK), jnp.float32),
            pltpu.SemaphoreType.DMA,
            pltpu.SemaphoreType.DMA,
            pltpu.SemaphoreType.DMA,
            pltpu.SemaphoreType.DMA,
        ],
        compiler_params=pltpu.CompilerParams(use_tc_tiling_on_sc=False,
                                             needs_layout_passes=False),
    )
    def emb(idx_hbm, table_hbm, out_hbm, idx_v, rows_a, rows_b,
            stage_a, stage_b, gsem_a, gsem_b, osem_a, osem_b):
        wid = lax.axis_index("s") * 2 + lax.axis_index("c")
        base_c = wid * nchunk
        pltpu.sync_copy(idx_hbm.at[pl.ds(base_c, nchunk)], idx_v)

        iota = lax.iota(jnp.int32, _LANES)

        def transpose_scale(rows, stage):
            @plsc.parallel_loop(0, 8, unroll=2)
            def tile_body(d_hi):
                for d_lo in range(8):
                    d_splat = jnp.broadcast_to(d_hi * 8 + d_lo, (_LANES,))
                    for k in range(8):
                        v = plsc.load_gather(
                            rows, [iota + k * _LANES, d_splat])
                        stage[d_hi, d_lo, pl.ds(k * _LANES, _LANES)] = (
                            v * _SCALE)

        def start_gather(c_local, rows, sem):
            return pltpu.async_copy(
                table_hbm.at[idx_v.at[c_local]], rows, sem)

        def start_out(c_local, stage, sem):
            c = base_c + c_local
            j = c // n_bhi
            b_hi = lax.rem(c, n_bhi)
            return pltpu.async_copy(stage, out_hbm.at[j, :, b_hi], sem)

        def wait_gather(rows, sem):
            pltpu.make_async_copy(table_hbm.at[pl.ds(0, _CHUNK)], rows,
                                  sem).wait()

        def wait_out(stage, sem):
            pltpu.make_async_copy(stage, out_hbm.at[0, :, 0], sem).wait()

        start_gather(0, rows_a, gsem_a)

        def body(t, carry):
            a = 2 * t
            b = 2 * t + 1
            wait_gather(rows_a, gsem_a)
            start_gather(b, rows_b, gsem_b)

            @pl.when(t > 0)
            def _():
                wait_out(stage_a, osem_a)
            transpose_scale(rows_a, stage_a)
            start_out(a, stage_a, osem_a)

            wait_gather(rows_b, gsem_b)

            @pl.when(t < nchunk // 2 - 1)
            def _():
                start_gather(b + 1, rows_a, gsem_a)

            @pl.when(t > 0)
            def _():
                wait_out(stage_b, osem_b)
            transpose_scale(rows_b, stage_b)
            start_out(b, stage_b, osem_b)
            return carry

        lax.fori_loop(0, nchunk // 2, body, 0)
        wait_out(stage_a, osem_a)
        wait_out(stage_b, osem_b)

    return emb


@jax.jit
def kernel(x, table):
    n_b, n_j = x.shape
    idx2 = x.T.reshape(n_j * (n_b // _CHUNK), _CHUNK)
    out5 = _make_kernel(n_b, n_j)(idx2, table)
    out = out5.transpose(2, 4, 0, 1, 3).reshape(n_b, n_j, _D)
    return out


# speedup vs baseline: 2.0870x; 1.4194x over previous
import functools

import jax
import jax.numpy as jnp
from jax import lax
from jax.experimental import pallas as pl
from jax.experimental.pallas import tpu as pltpu
from jax.experimental.pallas import tpu_sc as plsc

_D = 64
_SCALE = float(_D) ** 0.5
_NW = 32
_CHUNK = 128
_LANES = 16


def _make_kernel(n_b, n_j):
    n_bhi = n_b // _CHUNK
    nchunk = n_j * n_bhi // _NW
    mesh = plsc.VectorSubcoreMesh(core_axis_name="c", subcore_axis_name="s")

    @functools.partial(
        pl.kernel,
        mesh=mesh,
        out_type=jax.ShapeDtypeStruct((n_j, _D // 8, n_bhi, 8, _CHUNK),
                                      jnp.float32),
        scratch_types=[
            pltpu.VMEM((nchunk, _CHUNK), jnp.int32),
            pltpu.VMEM((_CHUNK, _D), jnp.float32),
            pltpu.VMEM((_CHUNK, _D), jnp.float32),
            pltpu.VMEM((_CHUNK, _D + 1), jnp.float32),
            pltpu.VMEM((_D // 8, 8, _CHUNK), jnp.float32),
            pltpu.VMEM((_D // 8, 8, _CHUNK), jnp.float32),
            pltpu.SemaphoreType.DMA,
            pltpu.SemaphoreType.DMA,
            pltpu.SemaphoreType.DMA,
            pltpu.SemaphoreType.DMA,
        ],
        compiler_params=pltpu.CompilerParams(use_tc_tiling_on_sc=False,
                                             needs_layout_passes=False),
    )
    def emb(idx_hbm, table_hbm, out_hbm, idx_v, rows_a, rows_b, rows_p,
            stage_a, stage_b, gsem_a, gsem_b, osem_a, osem_b):
        wid = lax.axis_index("s") * 2 + lax.axis_index("c")
        base_c = wid * nchunk
        pltpu.sync_copy(idx_hbm.at[pl.ds(base_c, nchunk)], idx_v)

        iota = lax.iota(jnp.int32, _LANES)

        def transpose_scale(rows, stage):
            @plsc.parallel_loop(0, _CHUNK, unroll=8)
            def copy_body(r):
                for k in range(_D // _LANES):
                    sl = pl.ds(k * _LANES, _LANES)
                    rows_p[r, sl] = rows[r, sl] * _SCALE

            @plsc.parallel_loop(0, 8, unroll=2)
            def tile_body(d_hi):
                for d_lo in range(8):
                    d_splat = jnp.broadcast_to(d_hi * 8 + d_lo, (_LANES,))
                    for k in range(8):
                        v = plsc.load_gather(
                            rows_p, [iota + k * _LANES, d_splat])
                        stage[d_hi, d_lo, pl.ds(k * _LANES, _LANES)] = v

        def start_gather(c_local, rows, sem):
            return pltpu.async_copy(
                table_hbm.at[idx_v.at[c_local]], rows, sem)

        def start_out(c_local, stage, sem):
            c = base_c + c_local
            j = c // n_bhi
            b_hi = lax.rem(c, n_bhi)
            return pltpu.async_copy(stage, out_hbm.at[j, :, b_hi], sem)

        def wait_gather(rows, sem):
            pltpu.make_async_copy(table_hbm.at[pl.ds(0, _CHUNK)], rows,
                                  sem).wait()

        def wait_out(stage, sem):
            pltpu.make_async_copy(stage, out_hbm.at[0, :, 0], sem).wait()

        start_gather(0, rows_a, gsem_a)

        def body(t, carry):
            a = 2 * t
            b = 2 * t + 1
            wait_gather(rows_a, gsem_a)
            start_gather(b, rows_b, gsem_b)

            @pl.when(t > 0)
            def _():
                wait_out(stage_a, osem_a)
            transpose_scale(rows_a, stage_a)
            start_out(a, stage_a, osem_a)

            wait_gather(rows_b, gsem_b)

            @pl.when(t < nchunk // 2 - 1)
            def _():
                start_gather(b + 1, rows_a, gsem_a)

            @pl.when(t > 0)
            def _():
                wait_out(stage_b, osem_b)
            transpose_scale(rows_b, stage_b)
            start_out(b, stage_b, osem_b)
            return carry

        lax.fori_loop(0, nchunk // 2, body, 0)
        wait_out(stage_a, osem_a)
        wait_out(stage_b, osem_b)

    return emb


@jax.jit
def kernel(x, table):
    n_b, n_j = x.shape
    idx2 = x.T.reshape(n_j * (n_b // _CHUNK), _CHUNK)
    out5 = _make_kernel(n_b, n_j)(idx2, table)
    out = out5.transpose(2, 4, 0, 1, 3).reshape(n_b, n_j, _D)
    return out


# speedup vs baseline: 2.6425x; 1.2662x over previous
import functools

import jax
import jax.numpy as jnp
from jax import lax
from jax.experimental import pallas as pl
from jax.experimental.pallas import tpu as pltpu
from jax.experimental.pallas import tpu_sc as plsc

_D = 64
_SCALE = float(_D) ** 0.5
_NW = 32
_CHUNK = 128
_LANES = 16


def _make_kernel(n_b, n_j, n_v):
    n_bhi = n_b // _CHUNK
    nchunk = n_j * n_bhi // _NW
    mesh = plsc.VectorSubcoreMesh(core_axis_name="c", subcore_axis_name="s")

    @functools.partial(
        pl.kernel,
        mesh=mesh,
        out_type=jax.ShapeDtypeStruct((n_j, _D // 8, n_bhi, 8, _CHUNK),
                                      jnp.float32),
        scratch_types=[
            pltpu.VMEM((nchunk, _CHUNK), jnp.int32),
            pltpu.VMEM((_CHUNK, _D), jnp.float32),
            pltpu.VMEM((_CHUNK, _D), jnp.float32),
            pltpu.VMEM((_D // 8, 8, _CHUNK + 1), jnp.float32),
            pltpu.VMEM((_D // 8, 8, _CHUNK + 1), jnp.float32),
            pltpu.SemaphoreType.DMA,
            pltpu.SemaphoreType.DMA,
            pltpu.SemaphoreType.DMA,
            pltpu.SemaphoreType.DMA,
        ],
        compiler_params=pltpu.CompilerParams(use_tc_tiling_on_sc=False,
                                             needs_layout_passes=False),
    )
    def emb(idx_hbm, table_hbm, out_hbm, idx_v, rows_a, rows_b,
            stage_a, stage_b, gsem_a, gsem_b, osem_a, osem_b):
        wid = lax.axis_index("s") * 2 + lax.axis_index("c")
        base_c = wid * nchunk
        pltpu.sync_copy(idx_hbm.at[pl.ds(base_c, nchunk)], idx_v)

        iota = lax.iota(jnp.int32, _LANES)
        d_hi_vecs = [(iota + m * _LANES) // 8 for m in range(_D // _LANES)]
        d_lo_vecs = [lax.rem(iota + m * _LANES, 8) for m in range(_D // _LANES)]

        def transpose_scale(rows, stage):
            @plsc.parallel_loop(0, _CHUNK, unroll=4)
            def row_body(r):
                r_splat = jnp.broadcast_to(r, (_LANES,))
                for m in range(_D // _LANES):
                    v = rows[r, pl.ds(m * _LANES, _LANES)] * _SCALE
                    plsc.store_scatter(
                        stage, [d_hi_vecs[m], d_lo_vecs[m], r_splat], v)

        def start_gather(c_local, rows, sem):
            return pltpu.async_copy(
                table_hbm.at[idx_v.at[c_local]], rows, sem)

        def start_out(c_local, stage, sem):
            c = base_c + c_local
            j = c // n_bhi
            b_hi = lax.rem(c, n_bhi)
            return pltpu.async_copy(stage.at[:, :, pl.ds(0, _CHUNK)],
                                    out_hbm.at[j, :, b_hi], sem)

        def wait_gather(rows, sem):
            pltpu.make_async_copy(table_hbm.at[pl.ds(0, _CHUNK)], rows,
                                  sem).wait()

        def wait_out(stage, sem):
            pltpu.make_async_copy(stage.at[:, :, pl.ds(0, _CHUNK)],
                                  out_hbm.at[0, :, 0], sem).wait()

        start_gather(0, rows_a, gsem_a)

        def body(t, carry):
            a = 2 * t
            b = 2 * t + 1
            wait_gather(rows_a, gsem_a)
            start_gather(b, rows_b, gsem_b)

            @pl.when(t > 0)
            def _():
                wait_out(stage_a, osem_a)
            transpose_scale(rows_a, stage_a)
            start_out(a, stage_a, osem_a)

            wait_gather(rows_b, gsem_b)

            @pl.when(t < nchunk // 2 - 1)
            def _():
                start_gather(b + 1, rows_a, gsem_a)

            @pl.when(t > 0)
            def _():
                wait_out(stage_b, osem_b)
            transpose_scale(rows_b, stage_b)
            start_out(b, stage_b, osem_b)
            return carry

        lax.fori_loop(0, nchunk // 2, body, 0)
        wait_out(stage_a, osem_a)
        wait_out(stage_b, osem_b)

    return emb


@jax.jit
def kernel(x, table):
    n_b, n_j = x.shape
    n_v = table.shape[0]
    idx2 = x.T.reshape(n_j * (n_b // _CHUNK), _CHUNK)
    out5 = _make_kernel(n_b, n_j, n_v)(idx2, table)
    out = out5.transpose(2, 4, 0, 1, 3).reshape(n_b, n_j, _D)
    return out
